# baseline (speedup 1.0000x reference)
import functools

import jax
import jax.numpy as jnp
from jax import lax
from jax.experimental import pallas as pl
from jax.experimental.pallas import tpu as pltpu
from jax.experimental.pallas import tpu_sc as plsc

_N, _E, _D, _RBF, _NRES = 10000, 320000, 128, 16, 3

_NC, _NS = 2, 16
_NW = _NC * _NS
_EPT = _E // _NW
_C = 80
_NF = _EPT // _C
_DB = 40
_NDB = _N // _DB

_BN = 1000
_BE = 4000


def _sp(x):
    return jnp.maximum(x, 0.0) + jnp.log(1.0 + jnp.exp(-jnp.abs(x)))


def _node_body(a_ref, wi_ref, bi_ref, wj_ref, bj_ref, u0_ref, y_ref):
    emb = _sp(a_ref[...])
    u0_ref[...] = _sp(
        jnp.dot(emb, wi_ref[...], preferred_element_type=jnp.float32) + bi_ref[...]
    )
    y_ref[...] = _sp(
        jnp.dot(emb, wj_ref[...], preferred_element_type=jnp.float32) + bj_ref[...]
    )


def _node_call(a, wi, bi, wj, bj):
    grid = (_N // _BN,)
    blk = pl.BlockSpec((_BN, _D), lambda i: (i, 0))
    wblk = pl.BlockSpec((_D, _D), lambda i: (0, 0))
    bblk = pl.BlockSpec((1, _D), lambda i: (0, 0))
    return pl.pallas_call(
        _node_body,
        grid=grid,
        in_specs=[blk, wblk, bblk, wblk, bblk],
        out_specs=[blk, blk],
        out_shape=[
            jax.ShapeDtypeStruct((_N, _D), jnp.float32),
            jax.ShapeDtypeStruct((_N, _D), jnp.float32),
        ],
    )(a, wi, bi, wj, bj)


def _g_body(f_ref, watt_ref, g_ref):
    g_ref[...] = jnp.dot(
        f_ref[...], watt_ref[...], preferred_element_type=jnp.float32
    )


def _g_call(f_ij, watt):
    grid = (_E // _BE,)
    return pl.pallas_call(
        _g_body,
        grid=grid,
        in_specs=[
            pl.BlockSpec((_BE, _RBF), lambda i: (i, 0)),
            pl.BlockSpec((_RBF, _D), lambda i: (0, 0)),
        ],
        out_specs=pl.BlockSpec((_BE, _D), lambda i: (i, 0)),
        out_shape=jax.ShapeDtypeStruct((_E, _D), jnp.float32),
    )(f_ij, watt)


_mesh = plsc.VectorSubcoreMesh(core_axis_name="c", subcore_axis_name="s")


@functools.partial(
    pl.kernel,
    out_type=jax.ShapeDtypeStruct((_NC, _N, _D), jnp.float32),
    mesh=_mesh,
    scratch_types=[
        pltpu.VMEM((_C,), jnp.int32),
        pltpu.VMEM((_C,), jnp.int32),
        pltpu.VMEM((_C,), jnp.int32),
        pltpu.VMEM((_C,), jnp.int32),
        pltpu.VMEM((_C, _D), jnp.float32),
        pltpu.VMEM((_C, _D), jnp.float32),
        pltpu.VMEM((_C, _D), jnp.float32),
        pltpu.VMEM((_C, _D), jnp.float32),
        pltpu.VMEM((_C,), jnp.int32),
        pltpu.VMEM((_C,), jnp.int32),
        pltpu.VMEM_SHARED((_N, _D), jnp.float32),
        pltpu.SemaphoreType.DMA,
        pltpu.SemaphoreType.DMA,
        pltpu.SemaphoreType.DMA,
        pltpu.SemaphoreType.DMA,
        pltpu.SemaphoreType.DMA,
        pltpu.SemaphoreType.DMA,
        pltpu.SemaphoreType.DMA,
        pltpu.SemaphoreType.DMA,
    ],
)
def _edge_kernel(y_hbm, g_hbm, ii_hbm, ij_hbm, out_hbm,
                 ij0, ij1, ii0, ii1, y0, y1, g0, g1,
                 sii0, sii1, agg,
                 si0, si1, sy0, sy1, sg0, sg1, ss0, ss1):
    c = lax.axis_index("c")
    s = lax.axis_index("s")
    t = c * _NS + s
    e0 = t * _EPT

    ij = (ij0, ij1)
    ii = (ii0, ii1)
    yb = (y0, y1)
    gb = (g0, g1)
    sii = (sii0, sii1)
    si = (si0, si1)
    sy = (sy0, sy1)
    sg = (sg0, sg1)
    ss = (ss0, ss1)

    zero = jnp.zeros((16,), jnp.float32)

    @plsc.parallel_loop(0, _DB, unroll=4)
    def _zrow(r):
        for v in range(_D // 16):
            y0[r, pl.ds(v * 16, 16)] = zero
    for kk in range(-(-_NDB // _NS)):
        b = s + kk * _NS

        @pl.when(b < _NDB)
        def _():
            pltpu.sync_copy(y0.at[pl.ds(0, _DB)], agg.at[pl.ds(b * _DB, _DB)])

    plsc.subcore_barrier()

    def issue_idx(ch, b):
        base = e0 + ch * _C
        pltpu.async_copy(ij_hbm.at[pl.ds(base, _C)], ij[b], si[b])
        pltpu.async_copy(ii_hbm.at[pl.ds(base, _C)], ii[b], si[b])

    def wait_idx(b):
        pltpu.make_async_copy(ij_hbm.at[pl.ds(0, _C)], ij[b], si[b]).wait()
        pltpu.make_async_copy(ii_hbm.at[pl.ds(0, _C)], ii[b], si[b]).wait()

    def issue_data(ch, b):
        base = e0 + ch * _C
        pltpu.async_copy(y_hbm.at[ij[b]], yb[b], sy[b])
        pltpu.async_copy(g_hbm.at[pl.ds(base, _C)], gb[b], sg[b])

    def wait_data(b):
        pltpu.make_async_copy(y_hbm.at[ij[b]], yb[b], sy[b]).wait()
        pltpu.make_async_copy(g_hbm.at[pl.ds(0, _C)], gb[b], sg[b]).wait()

    def mul_scatter(b):
        yr, gr = yb[b], gb[b]

        @plsc.parallel_loop(0, _C, unroll=8)
        def _mrow(r):
            for v in range(_D // 16):
                sl = pl.ds(v * 16, 16)
                yr[r, sl] = yr[r, sl] * gr[r, sl]

        @plsc.parallel_loop(0, _C // 16, unroll=5)
        def _crow(r):
            sii[b][pl.ds(r * 16, 16)] = ii[b][pl.ds(r * 16, 16)]

        pltpu.async_copy(yr, agg.at[sii[b]], ss[b], add=True)

    def wait_scatter(b):
        pltpu.make_async_copy(yb[b], agg.at[sii[b]], ss[b]).wait()

    pltpu.sync_copy(ij_hbm.at[pl.ds(e0, _C)], ij0)
    pltpu.sync_copy(ii_hbm.at[pl.ds(e0, _C)], ii0)
    issue_data(0, 0)
    issue_idx(1, 1)

    def pair_body(p, carry):
        for b in (0, 1):
            ch = 2 * p + b
            nb = 1 - b

            @pl.when(jnp.logical_and(ch >= 1, ch + 1 < _NF))
            def _():
                wait_scatter(nb)

            @pl.when(ch + 1 < _NF)
            def _():
                wait_idx(nb)
                issue_data(ch + 1, nb)

            wait_data(b)
            mul_scatter(b)

            @pl.when(ch + 2 < _NF)
            def _():
                issue_idx(ch + 2, b)

        return carry

    lax.fori_loop(0, _NF // 2, pair_body, 0)

    wait_data(0)
    mul_scatter(0)
    wait_scatter(1)
    wait_scatter(0)
    plsc.subcore_barrier()

    for kk in range(-(-_NDB // _NS)):
        b = s + kk * _NS

        @pl.when(b < _NDB)
        def _():
            pltpu.sync_copy(agg.at[pl.ds(b * _DB, _DB)], y0.at[pl.ds(0, _DB)])
            pltpu.sync_copy(y0.at[pl.ds(0, _DB)], out_hbm.at[c, pl.ds(b * _DB, _DB)])


def _final_body(a_ref, u0_ref, agg_ref, rw1_ref, rb1_ref, rw2_ref, rb2_ref,
                wv_ref, bv_ref, gate_ref, out_ref):
    emb = _sp(a_ref[...])
    u = u0_ref[...] + agg_ref[0] + agg_ref[1]
    for r in range(_NRES):
        h = jnp.dot(_sp(u), rw1_ref[r], preferred_element_type=jnp.float32)
        h = _sp(h + rb1_ref[r])
        u = u + jnp.dot(h, rw2_ref[r], preferred_element_type=jnp.float32) + rb2_ref[r]
    u = _sp(u)
    out_ref[...] = (
        gate_ref[...] * emb
        + jnp.dot(u, wv_ref[...], preferred_element_type=jnp.float32)
        + bv_ref[...]
    )


def _final_call(a, u0, agg, rw1, rb1, rw2, rb2, wv, bv, gate):
    grid = (_N // _BN,)
    blk = pl.BlockSpec((_BN, _D), lambda i: (i, 0))
    return pl.pallas_call(
        _final_body,
        grid=grid,
        in_specs=[
            blk,
            blk,
            pl.BlockSpec((_NC, _BN, _D), lambda i: (0, i, 0)),
            pl.BlockSpec((_NRES, _D, _D), lambda i: (0, 0, 0)),
            pl.BlockSpec((_NRES, 1, _D), lambda i: (0, 0, 0)),
            pl.BlockSpec((_NRES, _D, _D), lambda i: (0, 0, 0)),
            pl.BlockSpec((_NRES, 1, _D), lambda i: (0, 0, 0)),
            pl.BlockSpec((_D, _D), lambda i: (0, 0)),
            pl.BlockSpec((1, _D), lambda i: (0, 0)),
            pl.BlockSpec((1, _D), lambda i: (0, 0)),
        ],
        out_specs=blk,
        out_shape=jax.ShapeDtypeStruct((_N, _D), jnp.float32),
    )(a, u0, agg, rw1, rb1, rw2, rb2, wv, bv, gate)


def kernel(atomic_embedding, pair_indices, f_ij, W_att, W_i, b_i, W_j, b_j,
           W_v, b_v, res_W1, res_b1, res_W2, res_b2, gate):
    idx = pair_indices.astype(jnp.int32)
    idx_i = idx[0]
    idx_j = idx[1]

    u0, y = _node_call(atomic_embedding, W_i, b_i.reshape(1, _D),
                       W_j, b_j.reshape(1, _D))
    g = _g_call(f_ij, W_att)
    agg = _edge_kernel(y, g, idx_i, idx_j)
    out = _final_call(
        atomic_embedding, u0, agg,
        res_W1, res_b1.reshape(_NRES, 1, _D),
        res_W2, res_b2.reshape(_NRES, 1, _D),
        W_v, b_v.reshape(1, _D), gate.reshape(1, _D),
    )
    return out


# speedup vs baseline: 1.3895x; 1.3895x over previous
import functools

import jax
import jax.numpy as jnp
from jax import lax
from jax.experimental import pallas as pl
from jax.experimental.pallas import tpu as pltpu
from jax.experimental.pallas import tpu_sc as plsc

_N, _E, _D, _RBF, _NRES = 10000, 320000, 128, 16, 3

_NC, _NS = 2, 16
_NW = _NC * _NS
_EPT = _E // _NW
_C = 80
_NF = _EPT // _C
_DB = 40
_NDB = _N // _DB

_BN = 1000
_BE = 6400


def _sp(x):
    return jnp.maximum(x, 0.0) + jnp.log(1.0 + jnp.exp(-jnp.abs(x)))


def _node_body(a_ref, wi_ref, bi_ref, wj_ref, bj_ref, u0_ref, y_ref):
    emb = _sp(a_ref[...])
    u0_ref[...] = _sp(
        jnp.dot(emb, wi_ref[...], preferred_element_type=jnp.float32) + bi_ref[...]
    )
    y_ref[...] = _sp(
        jnp.dot(emb, wj_ref[...], preferred_element_type=jnp.float32) + bj_ref[...]
    )


def _node_call(a, wi, bi, wj, bj):
    grid = (_N // _BN,)
    blk = pl.BlockSpec((_BN, _D), lambda i: (i, 0))
    wblk = pl.BlockSpec((_D, _D), lambda i: (0, 0))
    bblk = pl.BlockSpec((1, _D), lambda i: (0, 0))
    return pl.pallas_call(
        _node_body,
        grid=grid,
        in_specs=[blk, wblk, bblk, wblk, bblk],
        out_specs=[blk, blk],
        out_shape=[
            jax.ShapeDtypeStruct((_N, _D), jnp.float32),
            jax.ShapeDtypeStruct((_N, _D), jnp.float32),
        ],
    )(a, wi, bi, wj, bj)


def _g_body(ft_ref, watt_ref, g_ref):
    g_ref[...] = lax.dot_general(
        ft_ref[...], watt_ref[...], (((0,), (0,)), ((), ())),
        preferred_element_type=jnp.float32,
    )


def _g_call(f_t, watt):
    grid = (_E // _BE,)
    return pl.pallas_call(
        _g_body,
        grid=grid,
        in_specs=[
            pl.BlockSpec((_RBF, _BE), lambda i: (0, i)),
            pl.BlockSpec((_RBF, _D), lambda i: (0, 0)),
        ],
        out_specs=pl.BlockSpec((_BE, _D), lambda i: (i, 0)),
        out_shape=jax.ShapeDtypeStruct((_E, _D), jnp.float32),
    )(f_t, watt)


_mesh = plsc.VectorSubcoreMesh(core_axis_name="c", subcore_axis_name="s")


@functools.partial(
    pl.kernel,
    out_type=jax.ShapeDtypeStruct((_NC, _N, _D), jnp.float32),
    mesh=_mesh,
    scratch_types=[
        pltpu.VMEM((_C,), jnp.int32),
        pltpu.VMEM((_C,), jnp.int32),
        pltpu.VMEM((_C,), jnp.int32),
        pltpu.VMEM((_C,), jnp.int32),
        pltpu.VMEM((_C, _D), jnp.float32),
        pltpu.VMEM((_C, _D), jnp.float32),
        pltpu.VMEM((_C, _D), jnp.float32),
        pltpu.VMEM((_C, _D), jnp.float32),
        pltpu.VMEM((_C,), jnp.int32),
        pltpu.VMEM((_C,), jnp.int32),
        pltpu.VMEM_SHARED((_N, _D), jnp.float32),
        pltpu.SemaphoreType.DMA,
        pltpu.SemaphoreType.DMA,
        pltpu.SemaphoreType.DMA,
        pltpu.SemaphoreType.DMA,
        pltpu.SemaphoreType.DMA,
        pltpu.SemaphoreType.DMA,
        pltpu.SemaphoreType.DMA,
        pltpu.SemaphoreType.DMA,
    ],
)
def _edge_kernel(y_hbm, g_hbm, ii_hbm, ij_hbm, out_hbm,
                 ij0, ij1, ii0, ii1, y0, y1, g0, g1,
                 sii0, sii1, agg,
                 si0, si1, sy0, sy1, sg0, sg1, ss0, ss1):
    c = lax.axis_index("c")
    s = lax.axis_index("s")
    t = c * _NS + s
    e0 = t * _EPT

    ij = (ij0, ij1)
    ii = (ii0, ii1)
    yb = (y0, y1)
    gb = (g0, g1)
    sii = (sii0, sii1)
    si = (si0, si1)
    sy = (sy0, sy1)
    sg = (sg0, sg1)
    ss = (ss0, ss1)

    zero = jnp.zeros((16,), jnp.float32)

    @plsc.parallel_loop(0, _DB, unroll=4)
    def _zrow(r):
        for v in range(_D // 16):
            y0[r, pl.ds(v * 16, 16)] = zero
    for kk in range(-(-_NDB // _NS)):
        b = s + kk * _NS

        @pl.when(b < _NDB)
        def _():
            pltpu.sync_copy(y0.at[pl.ds(0, _DB)], agg.at[pl.ds(b * _DB, _DB)])

    plsc.subcore_barrier()

    def issue_idx(ch, b):
        base = e0 + ch * _C
        pltpu.async_copy(ij_hbm.at[pl.ds(base, _C)], ij[b], si[b])
        pltpu.async_copy(ii_hbm.at[pl.ds(base, _C)], ii[b], si[b])

    def wait_idx(b):
        pltpu.make_async_copy(ij_hbm.at[pl.ds(0, _C)], ij[b], si[b]).wait()
        pltpu.make_async_copy(ii_hbm.at[pl.ds(0, _C)], ii[b], si[b]).wait()

    def issue_data(ch, b):
        base = e0 + ch * _C
        pltpu.async_copy(y_hbm.at[ij[b]], yb[b], sy[b])
        pltpu.async_copy(g_hbm.at[pl.ds(base, _C)], gb[b], sg[b])

    def wait_data(b):
        pltpu.make_async_copy(y_hbm.at[ij[b]], yb[b], sy[b]).wait()
        pltpu.make_async_copy(g_hbm.at[pl.ds(0, _C)], gb[b], sg[b]).wait()

    def mul_scatter(b):
        yr, gr = yb[b], gb[b]

        @plsc.parallel_loop(0, _C, unroll=4)
        def _mrow(r):
            for v in range(_D // 16):
                sl = pl.ds(v * 16, 16)
                yr[r, sl] = yr[r, sl] * gr[r, sl]

        @plsc.parallel_loop(0, _C // 16, unroll=5)
        def _crow(r):
            sii[b][pl.ds(r * 16, 16)] = ii[b][pl.ds(r * 16, 16)]

        pltpu.async_copy(yr, agg.at[sii[b]], ss[b], add=True)

    def wait_scatter(b):
        pltpu.make_async_copy(yb[b], agg.at[sii[b]], ss[b]).wait()

    pltpu.sync_copy(ij_hbm.at[pl.ds(e0, _C)], ij0)
    pltpu.sync_copy(ii_hbm.at[pl.ds(e0, _C)], ii0)
    issue_data(0, 0)
    issue_idx(1, 1)

    def pair_body(p, carry):
        for b in (0, 1):
            ch = 2 * p + b
            nb = 1 - b

            @pl.when(jnp.logical_and(ch >= 1, ch + 1 < _NF))
            def _():
                wait_scatter(nb)

            @pl.when(ch + 1 < _NF)
            def _():
                wait_idx(nb)
                issue_data(ch + 1, nb)

            wait_data(b)
            mul_scatter(b)

            @pl.when(ch + 2 < _NF)
            def _():
                issue_idx(ch + 2, b)

        return carry

    lax.fori_loop(0, _NF // 2, pair_body, 0)

    wait_data(0)
    mul_scatter(0)
    wait_scatter(1)
    wait_scatter(0)
    plsc.subcore_barrier()

    for kk in range(-(-_NDB // _NS)):
        b = s + kk * _NS

        @pl.when(b < _NDB)
        def _():
            pltpu.sync_copy(agg.at[pl.ds(b * _DB, _DB)], y0.at[pl.ds(0, _DB)])
            pltpu.sync_copy(y0.at[pl.ds(0, _DB)], out_hbm.at[c, pl.ds(b * _DB, _DB)])


def _final_body(a_ref, u0_ref, agg_ref, rw1_ref, rb1_ref, rw2_ref, rb2_ref,
                wv_ref, bv_ref, gate_ref, out_ref):
    emb = _sp(a_ref[...])
    u = u0_ref[...] + agg_ref[0] + agg_ref[1]
    for r in range(_NRES):
        h = jnp.dot(_sp(u), rw1_ref[r], preferred_element_type=jnp.float32)
        h = _sp(h + rb1_ref[r])
        u = u + jnp.dot(h, rw2_ref[r], preferred_element_type=jnp.float32) + rb2_ref[r]
    u = _sp(u)
    out_ref[...] = (
        gate_ref[...] * emb
        + jnp.dot(u, wv_ref[...], preferred_element_type=jnp.float32)
        + bv_ref[...]
    )


def _final_call(a, u0, agg, rw1, rb1, rw2, rb2, wv, bv, gate):
    grid = (_N // _BN,)
    blk = pl.BlockSpec((_BN, _D), lambda i: (i, 0))
    return pl.pallas_call(
        _final_body,
        grid=grid,
        in_specs=[
            blk,
            blk,
            pl.BlockSpec((_NC, _BN, _D), lambda i: (0, i, 0)),
            pl.BlockSpec((_NRES, _D, _D), lambda i: (0, 0, 0)),
            pl.BlockSpec((_NRES, 1, _D), lambda i: (0, 0, 0)),
            pl.BlockSpec((_NRES, _D, _D), lambda i: (0, 0, 0)),
            pl.BlockSpec((_NRES, 1, _D), lambda i: (0, 0, 0)),
            pl.BlockSpec((_D, _D), lambda i: (0, 0)),
            pl.BlockSpec((1, _D), lambda i: (0, 0)),
            pl.BlockSpec((1, _D), lambda i: (0, 0)),
        ],
        out_specs=blk,
        out_shape=jax.ShapeDtypeStruct((_N, _D), jnp.float32),
    )(a, u0, agg, rw1, rb1, rw2, rb2, wv, bv, gate)


def kernel(atomic_embedding, pair_indices, f_ij, W_att, W_i, b_i, W_j, b_j,
           W_v, b_v, res_W1, res_b1, res_W2, res_b2, gate):
    idx = pair_indices.astype(jnp.int32)

    u0, y = _node_call(atomic_embedding, W_i, b_i.reshape(1, _D),
                       W_j, b_j.reshape(1, _D))
    g = _g_call(f_ij.T, W_att)
    agg = _edge_kernel(y, g, idx[0], idx[1])
    out = _final_call(
        atomic_embedding, u0, agg,
        res_W1, res_b1.reshape(_NRES, 1, _D),
        res_W2, res_b2.reshape(_NRES, 1, _D),
        W_v, b_v.reshape(1, _D), gate.reshape(1, _D),
    )
    return out


# speedup vs baseline: 1.4487x; 1.0426x over previous
import functools

import jax
import jax.numpy as jnp
from jax import lax
from jax.experimental import pallas as pl
from jax.experimental.pallas import tpu as pltpu
from jax.experimental.pallas import tpu_sc as plsc

_N, _E, _D, _RBF, _NRES = 10000, 320000, 128, 16, 3

_NC, _NS = 2, 16
_NW = _NC * _NS
_EPT = _E // _NW
_C = 80
_NF = _EPT // _C
_DB = 40
_NDB = _N // _DB

_BN = 1000
_BE = 12800


def _sp(x):
    return jnp.maximum(x, 0.0) + jnp.log(1.0 + jnp.exp(-jnp.abs(x)))


def _node_body(a_ref, wi_ref, bi_ref, wj_ref, bj_ref, u0_ref, y_ref):
    emb = _sp(a_ref[...])
    u0_ref[...] = _sp(
        jnp.dot(emb, wi_ref[...], preferred_element_type=jnp.float32) + bi_ref[...]
    )
    y_ref[...] = _sp(
        jnp.dot(emb, wj_ref[...], preferred_element_type=jnp.float32) + bj_ref[...]
    )


def _node_call(a, wi, bi, wj, bj):
    grid = (_N // _BN,)
    blk = pl.BlockSpec((_BN, _D), lambda i: (i, 0))
    wblk = pl.BlockSpec((_D, _D), lambda i: (0, 0))
    bblk = pl.BlockSpec((1, _D), lambda i: (0, 0))
    return pl.pallas_call(
        _node_body,
        grid=grid,
        in_specs=[blk, wblk, bblk, wblk, bblk],
        out_specs=[blk, blk],
        out_shape=[
            jax.ShapeDtypeStruct((_N, _D), jnp.float32),
            jax.ShapeDtypeStruct((_N, _D), jnp.float32),
        ],
    )(a, wi, bi, wj, bj)


def _g_body(ft_ref, watt_ref, g_ref):
    g_ref[...] = lax.dot_general(
        ft_ref[...], watt_ref[...], (((0,), (0,)), ((), ())),
        preferred_element_type=jnp.float32,
    )


def _g_call(f_t, watt):
    grid = (_E // _BE,)
    return pl.pallas_call(
        _g_body,
        grid=grid,
        in_specs=[
            pl.BlockSpec((_RBF, _BE), lambda i: (0, i)),
            pl.BlockSpec((_RBF, _D), lambda i: (0, 0)),
        ],
        out_specs=pl.BlockSpec((_BE, _D), lambda i: (i, 0)),
        out_shape=jax.ShapeDtypeStruct((_E, _D), jnp.float32),
    )(f_t, watt)


_mesh = plsc.VectorSubcoreMesh(core_axis_name="c", subcore_axis_name="s")


@functools.partial(
    pl.kernel,
    out_type=jax.ShapeDtypeStruct((_NC, _N, _D), jnp.float32),
    mesh=_mesh,
    scratch_types=[
        pltpu.VMEM((_C,), jnp.int32),
        pltpu.VMEM((_C,), jnp.int32),
        pltpu.VMEM((_C,), jnp.int32),
        pltpu.VMEM((_C,), jnp.int32),
        pltpu.VMEM((_C, _D), jnp.float32),
        pltpu.VMEM((_C, _D), jnp.float32),
        pltpu.VMEM((_C, _D), jnp.float32),
        pltpu.VMEM((_C, _D), jnp.float32),
        pltpu.VMEM((_C,), jnp.int32),
        pltpu.VMEM((_C,), jnp.int32),
        pltpu.VMEM_SHARED((_N, _D), jnp.float32),
        pltpu.SemaphoreType.DMA,
        pltpu.SemaphoreType.DMA,
        pltpu.SemaphoreType.DMA,
        pltpu.SemaphoreType.DMA,
        pltpu.SemaphoreType.DMA,
        pltpu.SemaphoreType.DMA,
        pltpu.SemaphoreType.DMA,
        pltpu.SemaphoreType.DMA,
    ],
)
def _edge_kernel(y_hbm, g_hbm, ii_hbm, ij_hbm, out_hbm,
                 ij0, ij1, ii0, ii1, y0, y1, g0, g1,
                 sii0, sii1, agg,
                 si0, si1, sy0, sy1, sg0, sg1, ss0, ss1):
    c = lax.axis_index("c")
    s = lax.axis_index("s")
    t = c * _NS + s
    e0 = t * _EPT

    ij = (ij0, ij1)
    ii = (ii0, ii1)
    yb = (y0, y1)
    gb = (g0, g1)
    sii = (sii0, sii1)
    si = (si0, si1)
    sy = (sy0, sy1)
    sg = (sg0, sg1)
    ss = (ss0, ss1)

    zero = jnp.zeros((16,), jnp.float32)

    @plsc.parallel_loop(0, _DB, unroll=4)
    def _zrow(r):
        for v in range(_D // 16):
            y0[r, pl.ds(v * 16, 16)] = zero
    for kk in range(-(-_NDB // _NS)):
        b = s + kk * _NS

        @pl.when(b < _NDB)
        def _():
            pltpu.sync_copy(y0.at[pl.ds(0, _DB)], agg.at[pl.ds(b * _DB, _DB)])

    plsc.subcore_barrier()

    def issue_idx(ch, b):
        base = e0 + ch * _C
        pltpu.async_copy(ij_hbm.at[pl.ds(base, _C)], ij[b], si[b])
        pltpu.async_copy(ii_hbm.at[pl.ds(base, _C)], ii[b], si[b])

    def wait_idx(b):
        pltpu.make_async_copy(ij_hbm.at[pl.ds(0, _C)], ij[b], si[b]).wait()
        pltpu.make_async_copy(ii_hbm.at[pl.ds(0, _C)], ii[b], si[b]).wait()

    def issue_data(ch, b):
        base = e0 + ch * _C
        pltpu.async_copy(y_hbm.at[ij[b]], yb[b], sy[b])
        pltpu.async_copy(g_hbm.at[pl.ds(base, _C)], gb[b], sg[b])

    def wait_data(b):
        pltpu.make_async_copy(y_hbm.at[ij[b]], yb[b], sy[b]).wait()
        pltpu.make_async_copy(g_hbm.at[pl.ds(0, _C)], gb[b], sg[b]).wait()

    def mul_scatter(b):
        yr, gr = yb[b], gb[b]

        @plsc.parallel_loop(0, _C, unroll=4)
        def _mrow(r):
            for v in range(_D // 16):
                sl = pl.ds(v * 16, 16)
                yr[r, sl] = yr[r, sl] * gr[r, sl]

        @plsc.parallel_loop(0, _C // 16, unroll=5)
        def _crow(r):
            sii[b][pl.ds(r * 16, 16)] = ii[b][pl.ds(r * 16, 16)]

        pltpu.async_copy(yr, agg.at[sii[b]], ss[b], add=True)

    def wait_scatter(b):
        pltpu.make_async_copy(yb[b], agg.at[sii[b]], ss[b]).wait()

    pltpu.sync_copy(ij_hbm.at[pl.ds(e0, _C)], ij0)
    pltpu.sync_copy(ii_hbm.at[pl.ds(e0, _C)], ii0)
    issue_data(0, 0)
    issue_idx(1, 1)

    def pair_body(p, carry):
        for b in (0, 1):
            ch = 2 * p + b
            nb = 1 - b

            @pl.when(jnp.logical_and(ch >= 1, ch + 1 < _NF))
            def _():
                wait_scatter(nb)

            @pl.when(ch + 1 < _NF)
            def _():
                wait_idx(nb)
                issue_data(ch + 1, nb)

            wait_data(b)
            mul_scatter(b)

            @pl.when(ch + 2 < _NF)
            def _():
                issue_idx(ch + 2, b)

        return carry

    lax.fori_loop(0, _NF // 2, pair_body, 0)

    wait_data(0)
    mul_scatter(0)
    wait_scatter(1)
    wait_scatter(0)
    plsc.subcore_barrier()

    for kk in range(-(-_NDB // _NS)):
        b = s + kk * _NS

        @pl.when(b < _NDB)
        def _():
            pltpu.sync_copy(agg.at[pl.ds(b * _DB, _DB)], y0.at[pl.ds(0, _DB)])
            pltpu.sync_copy(y0.at[pl.ds(0, _DB)], out_hbm.at[c, pl.ds(b * _DB, _DB)])


def _final_body(a_ref, u0_ref, agg_ref, rw1_ref, rb1_ref, rw2_ref,
                rb2_ref, wv_ref, bv_ref, gate_ref, out_ref):
    emb = _sp(a_ref[...])
    u = u0_ref[...] + agg_ref[0] + agg_ref[1]
    for r in range(_NRES):
        h = jnp.dot(_sp(u), rw1_ref[r], preferred_element_type=jnp.float32)
        h = _sp(h + rb1_ref[r])
        u = u + jnp.dot(h, rw2_ref[r], preferred_element_type=jnp.float32) + rb2_ref[r]
    u = _sp(u)
    out_ref[...] = (
        gate_ref[...] * emb
        + jnp.dot(u, wv_ref[...], preferred_element_type=jnp.float32)
        + bv_ref[...]
    )


def _final_call(a, u0, agg, rw1, rb1, rw2, rb2, wv, bv, gate):
    grid = (_N // _BN,)
    blk = pl.BlockSpec((_BN, _D), lambda i: (i, 0))
    return pl.pallas_call(
        _final_body,
        grid=grid,
        in_specs=[
            blk,
            blk,
            pl.BlockSpec((_NC, _BN, _D), lambda i: (0, i, 0)),
            pl.BlockSpec((_NRES, _D, _D), lambda i: (0, 0, 0)),
            pl.BlockSpec((_NRES, 1, _D), lambda i: (0, 0, 0)),
            pl.BlockSpec((_NRES, _D, _D), lambda i: (0, 0, 0)),
            pl.BlockSpec((_NRES, 1, _D), lambda i: (0, 0, 0)),
            pl.BlockSpec((_D, _D), lambda i: (0, 0)),
            pl.BlockSpec((1, _D), lambda i: (0, 0)),
            pl.BlockSpec((1, _D), lambda i: (0, 0)),
        ],
        out_specs=blk,
        out_shape=jax.ShapeDtypeStruct((_N, _D), jnp.float32),
    )(a, u0, agg, rw1, rb1, rw2, rb2, wv, bv, gate)


def kernel(atomic_embedding, pair_indices, f_ij, W_att, W_i, b_i, W_j, b_j,
           W_v, b_v, res_W1, res_b1, res_W2, res_b2, gate):
    idx = pair_indices.astype(jnp.int32)

    u0, y = _node_call(atomic_embedding, W_i, b_i.reshape(1, _D),
                       W_j, b_j.reshape(1, _D))
    g = _g_call(f_ij.T, W_att)
    agg = _edge_kernel(y, g, idx[0], idx[1])
    out = _final_call(
        atomic_embedding, u0, agg,
        res_W1, res_b1.reshape(_NRES, 1, _D),
        res_W2, res_b2.reshape(_NRES, 1, _D),
        W_v, b_v.reshape(1, _D), gate.reshape(1, _D),
    )
    return out


# speedup vs baseline: 1.4607x; 1.0083x over previous
import functools

import jax
import jax.numpy as jnp
from jax import lax
from jax.experimental import pallas as pl
from jax.experimental.pallas import tpu as pltpu
from jax.experimental.pallas import tpu_sc as plsc

_N, _E, _D, _RBF, _NRES = 10000, 320000, 128, 16, 3

_NC, _NS = 2, 16
_NW = _NC * _NS
_EPT = _E // _NW
_C = 80
_NF = _EPT // _C
_DB = 40
_NDB = _N // _DB

_BN = 1000
_BE = 16000


def _sp(x):
    return jnp.maximum(x, 0.0) + jnp.log(1.0 + jnp.exp(-jnp.abs(x)))


def _node_body(a_ref, wi_ref, bi_ref, wj_ref, bj_ref, u0_ref, y_ref):
    emb = _sp(a_ref[...])
    u0_ref[...] = _sp(
        jnp.dot(emb, wi_ref[...], preferred_element_type=jnp.float32) + bi_ref[...]
    )
    y_ref[...] = _sp(
        jnp.dot(emb, wj_ref[...], preferred_element_type=jnp.float32) + bj_ref[...]
    )


def _node_call(a, wi, bi, wj, bj):
    grid = (_N // _BN,)
    blk = pl.BlockSpec((_BN, _D), lambda i: (i, 0))
    wblk = pl.BlockSpec((_D, _D), lambda i: (0, 0))
    bblk = pl.BlockSpec((1, _D), lambda i: (0, 0))
    return pl.pallas_call(
        _node_body,
        grid=grid,
        in_specs=[blk, wblk, bblk, wblk, bblk],
        out_specs=[blk, blk],
        out_shape=[
            jax.ShapeDtypeStruct((_N, _D), jnp.float32),
            jax.ShapeDtypeStruct((_N, _D), jnp.float32),
        ],
    )(a, wi, bi, wj, bj)


def _g_body(ft_ref, watt_ref, g_ref):
    g_ref[...] = lax.dot_general(
        ft_ref[...], watt_ref[...], (((0,), (0,)), ((), ())),
        preferred_element_type=jnp.float32,
    )


def _g_call(f_t, watt):
    grid = (_E // _BE,)
    return pl.pallas_call(
        _g_body,
        grid=grid,
        in_specs=[
            pl.BlockSpec((_RBF, _BE), lambda i: (0, i)),
            pl.BlockSpec((_RBF, _D), lambda i: (0, 0)),
        ],
        out_specs=pl.BlockSpec((_BE, _D), lambda i: (i, 0)),
        out_shape=jax.ShapeDtypeStruct((_E, _D), jnp.float32),
    )(f_t, watt)


_mesh = plsc.VectorSubcoreMesh(core_axis_name="c", subcore_axis_name="s")


@functools.partial(
    pl.kernel,
    out_type=jax.ShapeDtypeStruct((_NC, _N, _D), jnp.float32),
    mesh=_mesh,
    scratch_types=[
        pltpu.VMEM((_C,), jnp.int32),
        pltpu.VMEM((_C,), jnp.int32),
        pltpu.VMEM((_C,), jnp.int32),
        pltpu.VMEM((_C,), jnp.int32),
        pltpu.VMEM((_C, _D), jnp.float32),
        pltpu.VMEM((_C, _D), jnp.float32),
        pltpu.VMEM((_C, _D), jnp.float32),
        pltpu.VMEM((_C, _D), jnp.float32),
        pltpu.VMEM((_C,), jnp.int32),
        pltpu.VMEM((_C,), jnp.int32),
        pltpu.VMEM_SHARED((_N, _D), jnp.float32),
        pltpu.SemaphoreType.DMA,
        pltpu.SemaphoreType.DMA,
        pltpu.SemaphoreType.DMA,
        pltpu.SemaphoreType.DMA,
        pltpu.SemaphoreType.DMA,
        pltpu.SemaphoreType.DMA,
        pltpu.SemaphoreType.DMA,
        pltpu.SemaphoreType.DMA,
    ],
)
def _edge_kernel(y_hbm, g_hbm, ii_hbm, ij_hbm, out_hbm,
                 ij0, ij1, ii0, ii1, y0, y1, g0, g1,
                 sii0, sii1, agg,
                 si0, si1, sy0, sy1, sg0, sg1, ss0, ss1):
    c = lax.axis_index("c")
    s = lax.axis_index("s")
    t = c * _NS + s
    e0 = t * _EPT

    ij = (ij0, ij1)
    ii = (ii0, ii1)
    yb = (y0, y1)
    gb = (g0, g1)
    sii = (sii0, sii1)
    si = (si0, si1)
    sy = (sy0, sy1)
    sg = (sg0, sg1)
    ss = (ss0, ss1)

    zero = jnp.zeros((16,), jnp.float32)

    @plsc.parallel_loop(0, _DB, unroll=4)
    def _zrow(r):
        for v in range(_D // 16):
            y0[r, pl.ds(v * 16, 16)] = zero
    for kk in range(-(-_NDB // _NS)):
        b = s + kk * _NS

        @pl.when(b < _NDB)
        def _():
            pltpu.sync_copy(y0.at[pl.ds(0, _DB)], agg.at[pl.ds(b * _DB, _DB)])

    plsc.subcore_barrier()

    def issue_idx(ch, b):
        base = e0 + ch * _C
        pltpu.async_copy(ij_hbm.at[pl.ds(base, _C)], ij[b], si[b])
        pltpu.async_copy(ii_hbm.at[pl.ds(base, _C)], ii[b], si[b])

    def wait_idx(b):
        pltpu.make_async_copy(ij_hbm.at[pl.ds(0, _C)], ij[b], si[b]).wait()
        pltpu.make_async_copy(ii_hbm.at[pl.ds(0, _C)], ii[b], si[b]).wait()

    def issue_data(ch, b):
        base = e0 + ch * _C
        pltpu.async_copy(y_hbm.at[ij[b]], yb[b], sy[b])
        pltpu.async_copy(g_hbm.at[pl.ds(base, _C)], gb[b], sg[b])

    def wait_data(b):
        pltpu.make_async_copy(y_hbm.at[ij[b]], yb[b], sy[b]).wait()
        pltpu.make_async_copy(g_hbm.at[pl.ds(0, _C)], gb[b], sg[b]).wait()

    def mul_scatter(b):
        yr, gr = yb[b], gb[b]

        @plsc.parallel_loop(0, _C, unroll=4)
        def _mrow(r):
            for v in range(_D // 16):
                sl = pl.ds(v * 16, 16)
                yr[r, sl] = yr[r, sl] * gr[r, sl]

        @plsc.parallel_loop(0, _C // 16, unroll=5)
        def _crow(r):
            sii[b][pl.ds(r * 16, 16)] = ii[b][pl.ds(r * 16, 16)]

        pltpu.async_copy(yr, agg.at[sii[b]], ss[b], add=True)

    def wait_scatter(b):
        pltpu.make_async_copy(yb[b], agg.at[sii[b]], ss[b]).wait()

    pltpu.sync_copy(ij_hbm.at[pl.ds(e0, _C)], ij0)
    pltpu.sync_copy(ii_hbm.at[pl.ds(e0, _C)], ii0)
    issue_data(0, 0)
    issue_idx(1, 1)

    def pair_body(p, carry):
        for b in (0, 1):
            ch = 2 * p + b
            nb = 1 - b

            @pl.when(jnp.logical_and(ch >= 1, ch + 1 < _NF))
            def _():
                wait_scatter(nb)

            @pl.when(ch + 1 < _NF)
            def _():
                wait_idx(nb)
                issue_data(ch + 1, nb)

            wait_data(b)
            mul_scatter(b)

            @pl.when(ch + 2 < _NF)
            def _():
                issue_idx(ch + 2, b)

        return carry

    lax.fori_loop(0, _NF // 2, pair_body, 0)

    wait_data(0)
    mul_scatter(0)
    wait_scatter(1)
    wait_scatter(0)
    plsc.subcore_barrier()

    for kk in range(-(-_NDB // _NS)):
        b = s + kk * _NS

        @pl.when(b < _NDB)
        def _():
            pltpu.sync_copy(agg.at[pl.ds(b * _DB, _DB)], y0.at[pl.ds(0, _DB)])
            pltpu.sync_copy(y0.at[pl.ds(0, _DB)], out_hbm.at[c, pl.ds(b * _DB, _DB)])


def _final_body(a_ref, u0_ref, agg_ref, rw1_ref, rb1_ref, rw2_ref,
                rb2_ref, wv_ref, bv_ref, gate_ref, out_ref):
    emb = _sp(a_ref[...])
    u = u0_ref[...] + agg_ref[0] + agg_ref[1]
    for r in range(_NRES):
        h = jnp.dot(_sp(u), rw1_ref[r], preferred_element_type=jnp.float32)
        h = _sp(h + rb1_ref[r])
        u = u + jnp.dot(h, rw2_ref[r], preferred_element_type=jnp.float32) + rb2_ref[r]
    u = _sp(u)
    out_ref[...] = (
        gate_ref[...] * emb
        + jnp.dot(u, wv_ref[...], preferred_element_type=jnp.float32)
        + bv_ref[...]
    )


def _final_call(a, u0, agg, rw1, rb1, rw2, rb2, wv, bv, gate):
    grid = (_N // _BN,)
    blk = pl.BlockSpec((_BN, _D), lambda i: (i, 0))
    return pl.pallas_call(
        _final_body,
        grid=grid,
        in_specs=[
            blk,
            blk,
            pl.BlockSpec((_NC, _BN, _D), lambda i: (0, i, 0)),
            pl.BlockSpec((_NRES, _D, _D), lambda i: (0, 0, 0)),
            pl.BlockSpec((_NRES, 1, _D), lambda i: (0, 0, 0)),
            pl.BlockSpec((_NRES, _D, _D), lambda i: (0, 0, 0)),
            pl.BlockSpec((_NRES, 1, _D), lambda i: (0, 0, 0)),
            pl.BlockSpec((_D, _D), lambda i: (0, 0)),
            pl.BlockSpec((1, _D), lambda i: (0, 0)),
            pl.BlockSpec((1, _D), lambda i: (0, 0)),
        ],
        out_specs=blk,
        out_shape=jax.ShapeDtypeStruct((_N, _D), jnp.float32),
    )(a, u0, agg, rw1, rb1, rw2, rb2, wv, bv, gate)


def kernel(atomic_embedding, pair_indices, f_ij, W_att, W_i, b_i, W_j, b_j,
           W_v, b_v, res_W1, res_b1, res_W2, res_b2, gate):
    idx = pair_indices.astype(jnp.int32)

    u0, y = _node_call(atomic_embedding, W_i, b_i.reshape(1, _D),
                       W_j, b_j.reshape(1, _D))
    g = _g_call(f_ij.T, W_att)
    agg = _edge_kernel(y, g, idx[0], idx[1])
    out = _final_call(
        atomic_embedding, u0, agg,
        res_W1, res_b1.reshape(_NRES, 1, _D),
        res_W2, res_b2.reshape(_NRES, 1, _D),
        W_v, b_v.reshape(1, _D), gate.reshape(1, _D),
    )
    return out


# speedup vs baseline: 1.4609x; 1.0001x over previous
import functools

import jax
import jax.numpy as jnp
from jax import lax
from jax.experimental import pallas as pl
from jax.experimental.pallas import tpu as pltpu
from jax.experimental.pallas import tpu_sc as plsc

_N, _E, _D, _RBF, _NRES = 10000, 320000, 128, 16, 3

_NC, _NS = 2, 16
_NW = _NC * _NS
_EPT = _E // _NW
_C = 80
_NF = _EPT // _C
_DB = 40
_NDB = _N // _DB

_BN = 1000
_BE = 16000


def _sp(x):
    return jnp.maximum(x, 0.0) + jnp.log(1.0 + jnp.exp(-jnp.abs(x)))


def _node_body(a_ref, wi_ref, bi_ref, wj_ref, bj_ref, u0_ref, y_ref):
    emb = _sp(a_ref[...])
    u0_ref[...] = _sp(
        jnp.dot(emb, wi_ref[...], preferred_element_type=jnp.float32) + bi_ref[...]
    )
    y_ref[...] = _sp(
        jnp.dot(emb, wj_ref[...], preferred_element_type=jnp.float32) + bj_ref[...]
    )


def _node_call(a, wi, bi, wj, bj):
    grid = (_N // _BN,)
    blk = pl.BlockSpec((_BN, _D), lambda i: (i, 0))
    wblk = pl.BlockSpec((_D, _D), lambda i: (0, 0))
    bblk = pl.BlockSpec((1, _D), lambda i: (0, 0))
    return pl.pallas_call(
        _node_body,
        grid=grid,
        in_specs=[blk, wblk, bblk, wblk, bblk],
        out_specs=[blk, blk],
        out_shape=[
            jax.ShapeDtypeStruct((_N, _D), jnp.float32),
            jax.ShapeDtypeStruct((_N, _D), jnp.float32),
        ],
    )(a, wi, bi, wj, bj)


def _g_body(ft_ref, watt_ref, g_ref):
    g_ref[...] = lax.dot_general(
        ft_ref[...], watt_ref[...], (((0,), (0,)), ((), ())),
        preferred_element_type=jnp.float32,
    )


def _g_call(f_t, watt):
    grid = (_E // _BE,)
    return pl.pallas_call(
        _g_body,
        grid=grid,
        in_specs=[
            pl.BlockSpec((_RBF, _BE), lambda i: (0, i)),
            pl.BlockSpec((_RBF, _D), lambda i: (0, 0)),
        ],
        out_specs=pl.BlockSpec((_BE, _D), lambda i: (i, 0)),
        out_shape=jax.ShapeDtypeStruct((_E, _D), jnp.float32),
    )(f_t, watt)


_mesh = plsc.VectorSubcoreMesh(core_axis_name="c", subcore_axis_name="s")


@functools.partial(
    pl.kernel,
    out_type=jax.ShapeDtypeStruct((_NC, _N, _D), jnp.float32),
    mesh=_mesh,
    scratch_types=[
        pltpu.VMEM((_C,), jnp.int32),
        pltpu.VMEM((_C,), jnp.int32),
        pltpu.VMEM((_C,), jnp.int32),
        pltpu.VMEM((_C,), jnp.int32),
        pltpu.VMEM((_C, _D), jnp.float32),
        pltpu.VMEM((_C, _D), jnp.float32),
        pltpu.VMEM((_C, _D), jnp.float32),
        pltpu.VMEM((_C, _D), jnp.float32),
        pltpu.VMEM((_C,), jnp.int32),
        pltpu.VMEM((_C,), jnp.int32),
        pltpu.VMEM_SHARED((_N, _D), jnp.float32),
        pltpu.SemaphoreType.DMA,
        pltpu.SemaphoreType.DMA,
        pltpu.SemaphoreType.DMA,
        pltpu.SemaphoreType.DMA,
        pltpu.SemaphoreType.DMA,
        pltpu.SemaphoreType.DMA,
        pltpu.SemaphoreType.DMA,
        pltpu.SemaphoreType.DMA,
    ],
)
def _edge_kernel(y_hbm, g_hbm, ii_hbm, ij_hbm, out_hbm,
                 ij0, ij1, ii0, ii1, y0, y1, g0, g1,
                 sii0, sii1, agg,
                 si0, si1, sy0, sy1, sg0, sg1, ss0, ss1):
    c = lax.axis_index("c")
    s = lax.axis_index("s")
    t = c * _NS + s
    e0 = pl.multiple_of(t * _EPT, _C)

    ij = (ij0, ij1)
    ii = (ii0, ii1)
    yb = (y0, y1)
    gb = (g0, g1)
    sii = (sii0, sii1)
    si = (si0, si1)
    sy = (sy0, sy1)
    sg = (sg0, sg1)
    ss = (ss0, ss1)

    zero = jnp.zeros((16,), jnp.float32)

    @plsc.parallel_loop(0, _DB, unroll=4)
    def _zrow(r):
        for v in range(_D // 16):
            y0[r, pl.ds(v * 16, 16)] = zero
    for kk in range(-(-_NDB // _NS)):
        b = s + kk * _NS

        @pl.when(b < _NDB)
        def _():
            off = pl.multiple_of(b * _DB, _DB)
            pltpu.sync_copy(y0.at[pl.ds(0, _DB)], agg.at[pl.ds(off, _DB)])

    plsc.subcore_barrier()

    def issue_idx(ch, b):
        base = pl.multiple_of(e0 + ch * _C, _C)
        pltpu.async_copy(ij_hbm.at[pl.ds(base, _C)], ij[b], si[b])
        pltpu.async_copy(ii_hbm.at[pl.ds(base, _C)], ii[b], si[b])

    def wait_idx(b):
        pltpu.make_async_copy(ij_hbm.at[pl.ds(0, _C)], ij[b], si[b]).wait()
        pltpu.make_async_copy(ii_hbm.at[pl.ds(0, _C)], ii[b], si[b]).wait()

    def issue_data(ch, b):
        base = pl.multiple_of(e0 + ch * _C, _C)
        pltpu.async_copy(y_hbm.at[ij[b]], yb[b], sy[b])
        pltpu.async_copy(g_hbm.at[pl.ds(base, _C)], gb[b], sg[b])

    def wait_data(b):
        pltpu.make_async_copy(y_hbm.at[ij[b]], yb[b], sy[b]).wait()
        pltpu.make_async_copy(g_hbm.at[pl.ds(0, _C)], gb[b], sg[b]).wait()

    def mul_scatter(b):
        yr, gr = yb[b], gb[b]

        @plsc.parallel_loop(0, _C, unroll=4)
        def _mrow(r):
            for v in range(_D // 16):
                sl = pl.ds(v * 16, 16)
                yr[r, sl] = yr[r, sl] * gr[r, sl]

        @plsc.parallel_loop(0, _C // 16, unroll=5)
        def _crow(r):
            sii[b][pl.ds(r * 16, 16)] = ii[b][pl.ds(r * 16, 16)]

        pltpu.async_copy(yr, agg.at[sii[b]], ss[b], add=True)

    def wait_scatter(b):
        pltpu.make_async_copy(yb[b], agg.at[sii[b]], ss[b]).wait()

    pltpu.sync_copy(ij_hbm.at[pl.ds(e0, _C)], ij0)
    pltpu.sync_copy(ii_hbm.at[pl.ds(e0, _C)], ii0)
    issue_data(0, 0)
    issue_idx(1, 1)

    def pair_body(p, carry):
        for b in (0, 1):
            ch = 2 * p + b
            nb = 1 - b

            @pl.when(jnp.logical_and(ch >= 1, ch + 1 < _NF))
            def _():
                wait_scatter(nb)

            @pl.when(ch + 1 < _NF)
            def _():
                wait_idx(nb)
                issue_data(ch + 1, nb)

            wait_data(b)
            mul_scatter(b)

            @pl.when(ch + 2 < _NF)
            def _():
                issue_idx(ch + 2, b)

        return carry

    lax.fori_loop(0, _NF // 2, pair_body, 0)

    wait_data(0)
    mul_scatter(0)
    wait_scatter(1)
    wait_scatter(0)
    plsc.subcore_barrier()

    for kk in range(-(-_NDB // _NS)):
        b = s + kk * _NS

        @pl.when(b < _NDB)
        def _():
            off = pl.multiple_of(b * _DB, _DB)
            pltpu.sync_copy(agg.at[pl.ds(off, _DB)], y0.at[pl.ds(0, _DB)])
            pltpu.sync_copy(y0.at[pl.ds(0, _DB)], out_hbm.at[c, pl.ds(off, _DB)])


def _final_body(a_ref, u0_ref, agg_ref, rw1_ref, rb1_ref, rw2_ref,
                rb2_ref, wv_ref, bv_ref, gate_ref, out_ref):
    emb = _sp(a_ref[...])
    u = u0_ref[...] + agg_ref[0] + agg_ref[1]
    for r in range(_NRES):
        h = jnp.dot(_sp(u), rw1_ref[r], preferred_element_type=jnp.float32)
        h = _sp(h + rb1_ref[r])
        u = u + jnp.dot(h, rw2_ref[r], preferred_element_type=jnp.float32) + rb2_ref[r]
    u = _sp(u)
    out_ref[...] = (
        gate_ref[...] * emb
        + jnp.dot(u, wv_ref[...], preferred_element_type=jnp.float32)
        + bv_ref[...]
    )


def _final_call(a, u0, agg, rw1, rb1, rw2, rb2, wv, bv, gate):
    grid = (_N // _BN,)
    blk = pl.BlockSpec((_BN, _D), lambda i: (i, 0))
    return pl.pallas_call(
        _final_body,
        grid=grid,
        in_specs=[
            blk,
            blk,
            pl.BlockSpec((_NC, _BN, _D), lambda i: (0, i, 0)),
            pl.BlockSpec((_NRES, _D, _D), lambda i: (0, 0, 0)),
            pl.BlockSpec((_NRES, 1, _D), lambda i: (0, 0, 0)),
            pl.BlockSpec((_NRES, _D, _D), lambda i: (0, 0, 0)),
            pl.BlockSpec((_NRES, 1, _D), lambda i: (0, 0, 0)),
            pl.BlockSpec((_D, _D), lambda i: (0, 0)),
            pl.BlockSpec((1, _D), lambda i: (0, 0)),
            pl.BlockSpec((1, _D), lambda i: (0, 0)),
        ],
        out_specs=blk,
        out_shape=jax.ShapeDtypeStruct((_N, _D), jnp.float32),
    )(a, u0, agg, rw1, rb1, rw2, rb2, wv, bv, gate)


def kernel(atomic_embedding, pair_indices, f_ij, W_att, W_i, b_i, W_j, b_j,
           W_v, b_v, res_W1, res_b1, res_W2, res_b2, gate):
    idx = pair_indices.astype(jnp.int32)

    u0, y = _node_call(atomic_embedding, W_i, b_i.reshape(1, _D),
                       W_j, b_j.reshape(1, _D))
    g = _g_call(f_ij.T, W_att)
    agg = _edge_kernel(y, g, idx[0], idx[1])
    out = _final_call(
        atomic_embedding, u0, agg,
        res_W1, res_b1.reshape(_NRES, 1, _D),
        res_W2, res_b2.reshape(_NRES, 1, _D),
        W_v, b_v.reshape(1, _D), gate.reshape(1, _D),
    )
    return out
